# baseline (speedup 1.0000x reference)
import jax
import jax.numpy as jnp
from jax.experimental import pallas as pl

_BLOCK_ROWS = 8192


def _copy_body(u_ref, i_ref, ou_ref, oi_ref):
    ou_ref[...] = u_ref[...]
    oi_ref[...] = i_ref[...]


def kernel(user_emb, item_emb):
    n_u, d = user_emb.shape
    n_i, _ = item_emb.shape
    grid = (pl.cdiv(max(n_u, n_i), _BLOCK_ROWS),)
    out_u, out_i = pl.pallas_call(
        _copy_body,
        grid=grid,
        in_specs=[
            pl.BlockSpec((_BLOCK_ROWS, d), lambda r: (r, 0)),
            pl.BlockSpec((_BLOCK_ROWS, d), lambda r: (r, 0)),
        ],
        out_specs=[
            pl.BlockSpec((_BLOCK_ROWS, d), lambda r: (r, 0)),
            pl.BlockSpec((_BLOCK_ROWS, d), lambda r: (r, 0)),
        ],
        out_shape=[
            jax.ShapeDtypeStruct((n_u, d), user_emb.dtype),
            jax.ShapeDtypeStruct((n_i, d), item_emb.dtype),
        ],
    )(user_emb, item_emb)
    return (out_u, out_i)
